# Initial kernel scaffold; baseline (speedup 1.0000x reference)
import jax
import jax.numpy as jnp
from jax.experimental import pallas as pl


def kernel(predictions, x, labels, classCenter):
    raise NotImplementedError("write your pallas kernel here")


# speedup vs baseline: 1.6612x; 1.6612x over previous
import functools

import jax
import jax.numpy as jnp
from jax.experimental import pallas as pl
from jax.experimental.pallas import tpu as pltpu

LABELS = 1000
FEATURES = 512
LAMBDA1 = 0.01
BM = 2048


def _body(p_ref, x_ref, lab_ref, cc_ref, out_ref, acc_ref):
    i = pl.program_id(0)
    p = p_ref[...]
    m = jnp.max(p, axis=1, keepdims=True)
    s = jnp.sum(jnp.exp(p - m), axis=1, keepdims=True)
    lse_sum = jnp.sum(m + jnp.log(s))

    lab = lab_ref[0]
    col = jax.lax.broadcasted_iota(jnp.int32, (BM, LABELS), 1)
    mask = col == lab
    picked_sum = jnp.sum(jnp.where(mask, p, 0.0))
    counts = jnp.sum(mask.astype(jnp.float32), axis=0, keepdims=True)
    gathered = jnp.dot(counts, cc_ref[...], preferred_element_type=jnp.float32)
    rs_sum = jnp.sum(gathered)

    xs = jnp.sum(x_ref[...])

    part_a = lse_sum - picked_sum
    part_b = xs - rs_sum

    @pl.when(i == 0)
    def _init():
        acc_ref[0] = part_a
        acc_ref[1] = part_b

    @pl.when(i > 0)
    def _acc():
        acc_ref[0] += part_a
        acc_ref[1] += part_b

    @pl.when(i == pl.num_programs(0) - 1)
    def _fin():
        out_ref[0, 0] = acc_ref[0] + (LAMBDA1 / 2.0) * acc_ref[1] * acc_ref[1]


@functools.partial(jax.jit, static_argnames=())
def kernel(predictions, x, labels, classCenter):
    batch = predictions.shape[0]
    grid = batch // BM
    lab3 = labels.astype(jnp.int32).reshape(grid, BM, 1)
    out = pl.pallas_call(
        _body,
        grid=(grid,),
        in_specs=[
            pl.BlockSpec((BM, LABELS), lambda i: (i, 0)),
            pl.BlockSpec((BM, FEATURES), lambda i: (i, 0)),
            pl.BlockSpec((1, BM, 1), lambda i: (i, 0, 0)),
            pl.BlockSpec((LABELS, FEATURES), lambda i: (0, 0)),
        ],
        out_specs=pl.BlockSpec((1, 1), lambda i: (0, 0), memory_space=pltpu.SMEM),
        out_shape=jax.ShapeDtypeStruct((1, 1), jnp.float32),
        scratch_shapes=[pltpu.SMEM((2,), jnp.float32)],
        compiler_params=pltpu.CompilerParams(
            dimension_semantics=("arbitrary",),
        ),
    )(predictions, x, lab3, classCenter)
    return out.reshape(())
